# Initial kernel scaffold; baseline (speedup 1.0000x reference)
import jax
import jax.numpy as jnp
from jax.experimental import pallas as pl


def kernel(x, y):
    raise NotImplementedError("write your pallas kernel here")


# speedup vs baseline: 3.0724x; 3.0724x over previous
import jax
import jax.numpy as jnp
from jax.experimental import pallas as pl
from jax.experimental.pallas import tpu as pltpu

_GAMMA = 1.0


def _lm_body(y_ref, x_ref, o_ref, m_ref, c_ref, *, bc, ncols, nsteps, nrows):
    c = pl.program_id(0)

    @pl.when(c == 0)
    def _init():
        m_ref[...] = jnp.full((nrows, 1), -jnp.inf, dtype=jnp.float32)
        c_ref[...] = jnp.zeros((nrows, 1), dtype=jnp.float32)

    xb = x_ref[...]
    col_ids = c * bc + jax.lax.broadcasted_iota(jnp.int32, (nrows, bc), 1)
    eq = col_ids == y_ref[...]
    bad = eq | (col_ids >= ncols)
    masked = jnp.where(bad, -jnp.inf, xb)
    m_ref[...] = jnp.maximum(m_ref[...], jnp.max(masked, axis=1, keepdims=True))
    c_ref[...] = c_ref[...] + jnp.sum(jnp.where(eq, xb, 0.0), axis=1, keepdims=True)

    @pl.when(c == nsteps - 1)
    def _fin():
        loss = _GAMMA + m_ref[...] - c_ref[...]
        o_ref[...] = (jnp.sum(loss) * (1.0 / nrows)).reshape(1, 1)


def kernel(x, y):
    nrows, ncols = x.shape
    bc = 4096 if ncols >= 4096 else ncols
    nsteps = pl.cdiv(ncols, bc)
    y2 = y.astype(jnp.int32).reshape(nrows, 1)

    import functools
    body = functools.partial(
        _lm_body, bc=bc, ncols=ncols, nsteps=nsteps, nrows=nrows
    )
    out = pl.pallas_call(
        body,
        grid=(nsteps,),
        in_specs=[
            pl.BlockSpec((nrows, 1), lambda c: (0, 0)),
            pl.BlockSpec((nrows, bc), lambda c: (0, c)),
        ],
        out_specs=pl.BlockSpec((1, 1), lambda c: (0, 0)),
        out_shape=jax.ShapeDtypeStruct((1, 1), jnp.float32),
        scratch_shapes=[
            pltpu.VMEM((nrows, 1), jnp.float32),
            pltpu.VMEM((nrows, 1), jnp.float32),
        ],
        compiler_params=pltpu.CompilerParams(
            dimension_semantics=("arbitrary",),
        ),
    )(y2, x)
    return out[0, 0]
